# baseline (speedup 1.0000x reference)
import jax
import jax.numpy as jnp
from jax import lax
from jax.experimental import pallas as pl
from jax.experimental.pallas import tpu as pltpu
from jax.experimental.pallas import tpu_sc as plsc

_N_BONDS = 1600000
_N_TORS = 800000
_CONFS = 16
_NB = 64
_PER = 6
_NW = 32
_BONDS_PER_W = _N_BONDS // _NW
_TORS_PER_W = _N_TORS // _NW
_BC = 2000
_TCH = 1000
_NBC = _BONDS_PER_W // _BC
_NTC = _TORS_PER_W // _TCH
_G = 16

_COS_C = (
    9.99999992e-01,
    -4.99999918e-01,
    4.16665243e-02,
    -1.38879703e-03,
    2.47734208e-05,
    -2.71133377e-07,
    1.73689961e-09,
)
_INV_2PI = 0.15915494309189535
_PI2_HI = 6.28125
_PI2_LO = 0.0019353071795864769
_RND_MAGIC = 12582912.0


def _cos_stage(phis):
    a0, a1, a2, a3, a4, a5, a6 = _COS_C
    rns = [p * _INV_2PI for p in phis]
    nfs = [(rn + _RND_MAGIC) - _RND_MAGIC for rn in rns]
    rs = [(p - nf * _PI2_HI) - nf * _PI2_LO for p, nf in zip(phis, nfs)]
    ts = [r * r for r in rs]
    t2s = [t * t for t in ts]
    p01s = [a0 + a1 * t for t in ts]
    p23s = [a2 + a3 * t for t in ts]
    p45s = [a4 + a5 * t for t in ts]
    qs = [p45 + a6 * t2 for p45, t2 in zip(p45s, t2s)]
    lows = [p01 + t2 * p23 for p01, t2, p23 in zip(p01s, t2s, p23s)]
    return [low + (t2 * t2) * q for low, t2, q in zip(lows, t2s, qs)]


def _tree_sum(vs):
    vs = list(vs)
    while len(vs) > 1:
        vs = [vs[i] + vs[i + 1] for i in range(0, len(vs) - 1, 2)] \
            + ([vs[-1]] if len(vs) % 2 else [])
    return vs[0]


def _zero_acc(acc_v):
    zeros16 = jnp.zeros((_G,), jnp.float32)
    for i in range(_NB):
        acc_v[pl.ds(i * _G, _G)] = zeros16


def _scatter_window(acc_v, lanes, b16, j0, es):
    e_sum = _tree_sum(es)
    uniform = b16[j0] == b16[_G - 1]

    @pl.when(uniform)
    def _():
        idx = jnp.full((_G,), b16[j0]) * _G + lanes
        plsc.addupdate_scatter(acc_v, [idx], e_sum)

    @pl.when(jnp.logical_not(uniform))
    def _():
        for i, e in enumerate(es):
            idx = jnp.full((_G,), b16[j0 + i]) * _G + lanes
            plsc.addupdate_scatter(acc_v, [idx], e)


def _bond_body(kb_hbm, eq_hbm, d_hbm, bb_hbm, out_hbm,
               kb_v, eq_v, d_v, bb_v, acc_v):
    cid = lax.axis_index("c")
    sid = lax.axis_index("s")
    wid = sid * 2 + cid
    lanes = lax.iota(jnp.int32, _G)
    _zero_acc(acc_v)
    bond_base = wid * _BONDS_PER_W

    def chunk(ci, carry):
        base = bond_base + ci * _BC
        pltpu.sync_copy(kb_hbm.at[pl.ds(base, _BC)], kb_v)
        pltpu.sync_copy(eq_hbm.at[pl.ds(base, _BC)], eq_v)
        pltpu.sync_copy(d_hbm.at[pl.ds(base * _CONFS, _BC * _CONFS)], d_v)
        pltpu.sync_copy(bb_hbm.at[pl.ds(base, _BC)], bb_v)

        def window(g, c2):
            r0 = g * _G
            k16 = kb_v[pl.ds(r0, _G)]
            eq16 = eq_v[pl.ds(r0, _G)]
            b16 = bb_v[pl.ds(r0, _G)]
            ds = [d_v[pl.ds((r0 + j) * _CONFS, _G)] for j in range(_G)]
            diffs = [d - eq16[j] for j, d in enumerate(ds)]
            es = [(0.5 * k16[j]) * (df * df) for j, df in enumerate(diffs)]
            _scatter_window(acc_v, lanes, b16, 0, es)
            return c2
        return lax.fori_loop(0, _BC // _G, window, carry)

    lax.fori_loop(0, _NBC, chunk, 0)
    pltpu.sync_copy(acc_v, out_hbm.at[wid])


def _tors_body(ktf_hbm, ang_hbm, tb_hbm, out_hbm,
               ktf_v, ang_v, tb_v, acc_v):
    cid = lax.axis_index("c")
    sid = lax.axis_index("s")
    wid = sid * 2 + cid
    lanes = lax.iota(jnp.int32, _G)
    _zero_acc(acc_v)
    tors_base = wid * _TORS_PER_W

    def chunk(ci, carry):
        base = tors_base + ci * _TCH
        pltpu.sync_copy(ktf_hbm.at[pl.ds(base * _PER, _TCH * _PER)], ktf_v)
        pltpu.sync_copy(ang_hbm.at[pl.ds(base * _CONFS, _TCH * _CONFS)], ang_v)
        pltpu.sync_copy(tb_hbm.at[pl.ds(base, _TCH)], tb_v)

        def do_window(r0, j0):
            b16 = tb_v[pl.ds(r0, _G)]
            kt = [ktf_v[pl.ds(r0 * _PER // _G * _G + m * _G, _G)]
                  for m in range(_G * _PER // _G)]

            def kcoef(j, n):
                pos = _PER * j + n
                return kt[pos // _G][pos % _G]

            es = []
            half = (_G - j0) // 2
            rows = list(range(j0, _G))
            for batch in (rows[:half], rows[half:]):
                phis = [ang_v[pl.ds((r0 + j) * _CONFS, _G)] for j in batch]
                c1s = _cos_stage(phis)
                e_b = [kcoef(j, 0) * c1 for j, c1 in zip(batch, c1s)]
                cpps = c1s
                cps = [2.0 * c1 * c1 - 1.0 for c1 in c1s]
                e_b = [e + kcoef(j, 1) * cp
                       for e, j, cp in zip(e_b, batch, cps)]
                for n in range(2, _PER):
                    cns = [2.0 * c1 * cp - cpp
                           for c1, cp, cpp in zip(c1s, cps, cpps)]
                    e_b = [e + kcoef(j, n) * cn
                           for e, j, cn in zip(e_b, batch, cns)]
                    cpps = cps
                    cps = cns
                es.extend(e_b)
            _scatter_window(acc_v, lanes, b16, j0, es)

        def window(g, c2):
            do_window(g * _G, 0)
            return c2
        carry = lax.fori_loop(0, _TCH // _G, window, carry)
        if _TCH % _G:
            do_window(_TCH - _G, _G - _TCH % _G)
        return carry

    lax.fori_loop(0, _NTC, chunk, 0)
    pltpu.sync_copy(acc_v, out_hbm.at[wid])


def _combine_body(p1_ref, p2_ref, o_ref):
    o_ref[...] = jnp.sum(p1_ref[...], axis=0) + jnp.sum(p2_ref[...], axis=0)


@jax.jit
def kernel(k_bond, eq_bond, distances, bond_batch, k_torsion, angles, torsion_batch):
    mesh = plsc.VectorSubcoreMesh(core_axis_name="c", subcore_axis_name="s")
    cp = pltpu.CompilerParams(
        needs_layout_passes=False, use_tc_tiling_on_sc=False)
    out_t = jax.ShapeDtypeStruct((_NW, _NB * _CONFS), jnp.float32)
    bond_sc = pl.kernel(
        _bond_body, out_type=out_t, mesh=mesh, compiler_params=cp,
        scratch_types=[
            pltpu.VMEM((_BC,), jnp.float32),
            pltpu.VMEM((_BC,), jnp.float32),
            pltpu.VMEM((_BC * _CONFS,), jnp.float32),
            pltpu.VMEM((_BC,), jnp.int32),
            pltpu.VMEM((_NB * _CONFS,), jnp.float32),
        ],
    )
    tors_sc = pl.kernel(
        _tors_body, out_type=out_t, mesh=mesh, compiler_params=cp,
        scratch_types=[
            pltpu.VMEM((_TCH * _PER,), jnp.float32),
            pltpu.VMEM((_TCH * _CONFS,), jnp.float32),
            pltpu.VMEM((_TCH,), jnp.int32),
            pltpu.VMEM((_NB * _CONFS,), jnp.float32),
        ],
    )
    one = k_bond[0] * 0.0 + 1.0
    p_bond = bond_sc(k_bond, eq_bond, (distances * one).reshape(-1),
                     bond_batch)
    p_tors = tors_sc((k_torsion * one).reshape(-1),
                     (angles * one).reshape(-1), torsion_batch)
    total = pl.pallas_call(
        _combine_body,
        out_shape=jax.ShapeDtypeStruct((_NB, _CONFS), jnp.float32),
    )(p_bond.reshape(_NW, _NB, _CONFS), p_tors.reshape(_NW, _NB, _CONFS))
    return total


# speedup vs baseline: 1.0444x; 1.0444x over previous
import jax
import jax.numpy as jnp
from jax import lax
from jax.experimental import pallas as pl
from jax.experimental.pallas import tpu as pltpu
from jax.experimental.pallas import tpu_sc as plsc

_N_BONDS = 1600000
_N_TORS = 800000
_CONFS = 16
_NB = 64
_PER = 6
_NW = 32
_BONDS_PER_W = _N_BONDS // _NW
_TORS_PER_W = _N_TORS // _NW
_BC = 2000
_TCH = 1000
_NBC = _BONDS_PER_W // _BC
_NTC = _TORS_PER_W // _TCH
_G = 16

_COS_C = (
    9.99999992e-01,
    -4.99999918e-01,
    4.16665243e-02,
    -1.38879703e-03,
    2.47734208e-05,
    -2.71133377e-07,
    1.73689961e-09,
)
_INV_2PI = 0.15915494309189535
_PI2_HI = 6.28125
_PI2_LO = 0.0019353071795864769
_RND_MAGIC = 12582912.0


def _cos_stage(phis):
    a0, a1, a2, a3, a4, a5, a6 = _COS_C
    rns = [p * _INV_2PI for p in phis]
    nfs = [(rn + _RND_MAGIC) - _RND_MAGIC for rn in rns]
    rs = [(p - nf * _PI2_HI) - nf * _PI2_LO for p, nf in zip(phis, nfs)]
    ts = [r * r for r in rs]
    t2s = [t * t for t in ts]
    p01s = [a0 + a1 * t for t in ts]
    p23s = [a2 + a3 * t for t in ts]
    p45s = [a4 + a5 * t for t in ts]
    qs = [p45 + a6 * t2 for p45, t2 in zip(p45s, t2s)]
    lows = [p01 + t2 * p23 for p01, t2, p23 in zip(p01s, t2s, p23s)]
    return [low + (t2 * t2) * q for low, t2, q in zip(lows, t2s, qs)]


def _tree_sum(vs):
    vs = list(vs)
    while len(vs) > 1:
        vs = [vs[i] + vs[i + 1] for i in range(0, len(vs) - 1, 2)] \
            + ([vs[-1]] if len(vs) % 2 else [])
    return vs[0]


def _zero_acc(acc_v):
    zeros16 = jnp.zeros((_G,), jnp.float32)
    for i in range(_NB):
        acc_v[pl.ds(i * _G, _G)] = zeros16


def _scatter_window(acc_v, lanes, b16, j0, es):
    e_sum = _tree_sum(es)
    uniform = b16[j0] == b16[_G - 1]

    @pl.when(uniform)
    def _():
        idx = jnp.full((_G,), b16[j0]) * _G + lanes
        plsc.addupdate_scatter(acc_v, [idx], e_sum)

    @pl.when(jnp.logical_not(uniform))
    def _():
        for i, e in enumerate(es):
            idx = jnp.full((_G,), b16[j0 + i]) * _G + lanes
            plsc.addupdate_scatter(acc_v, [idx], e)


def _bond_body(kb_hbm, eq_hbm, d_hbm, bb_hbm, out_hbm,
               kb_v, eq_v, d_v, bb_v, acc_v):
    cid = lax.axis_index("c")
    sid = lax.axis_index("s")
    wid = sid * 2 + cid
    lanes = lax.iota(jnp.int32, _G)
    _zero_acc(acc_v)
    bond_base = wid * _BONDS_PER_W

    def chunk(ci, carry):
        base = bond_base + ci * _BC
        pltpu.sync_copy(kb_hbm.at[pl.ds(base, _BC)], kb_v)
        pltpu.sync_copy(eq_hbm.at[pl.ds(base, _BC)], eq_v)
        pltpu.sync_copy(d_hbm.at[pl.ds(base * _CONFS, _BC * _CONFS)], d_v)
        pltpu.sync_copy(bb_hbm.at[pl.ds(base, _BC)], bb_v)

        def window(g, c2):
            r0 = g * _G
            b16 = bb_v[pl.ds(r0, _G)]
            rsplat = jnp.full((_G,), r0, jnp.int32)
            kvs = [plsc.load_gather(kb_v, [rsplat + j]) for j in range(_G)]
            eqvs = [plsc.load_gather(eq_v, [rsplat + j]) for j in range(_G)]
            ds = [d_v[pl.ds((r0 + j) * _CONFS, _G)] for j in range(_G)]
            diffs = [d - eqv for d, eqv in zip(ds, eqvs)]
            es = [kv * (df * df) for kv, df in zip(kvs, diffs)]
            _scatter_window(acc_v, lanes, b16, 0, es)
            return c2
        return lax.fori_loop(0, _BC // _G, window, carry)

    lax.fori_loop(0, _NBC, chunk, 0)
    pltpu.sync_copy(acc_v, out_hbm.at[wid])


def _tors_body(ktf_hbm, ang_hbm, tb_hbm, out_hbm,
               ktf_v, ang_v, tb_v, acc_v):
    cid = lax.axis_index("c")
    sid = lax.axis_index("s")
    wid = sid * 2 + cid
    lanes = lax.iota(jnp.int32, _G)
    _zero_acc(acc_v)
    tors_base = wid * _TORS_PER_W

    def chunk(ci, carry):
        base = tors_base + ci * _TCH
        pltpu.sync_copy(ktf_hbm.at[pl.ds(base * _PER, _TCH * _PER)], ktf_v)
        pltpu.sync_copy(ang_hbm.at[pl.ds(base * _CONFS, _TCH * _CONFS)], ang_v)
        pltpu.sync_copy(tb_hbm.at[pl.ds(base, _TCH)], tb_v)

        def do_window(r0, j0):
            b16 = tb_v[pl.ds(r0, _G)]
            kbase = jnp.full((_G,), r0 * _PER, jnp.int32)

            def kcoef(j, n):
                return plsc.load_gather(ktf_v, [kbase + (_PER * j + n)])

            es = []
            half = (_G - j0) // 2
            rows = list(range(j0, _G))
            for batch in (rows[:half], rows[half:]):
                phis = [ang_v[pl.ds((r0 + j) * _CONFS, _G)] for j in batch]
                c1s = _cos_stage(phis)
                e_b = [kcoef(j, 0) * c1 for j, c1 in zip(batch, c1s)]
                cpps = c1s
                cps = [2.0 * c1 * c1 - 1.0 for c1 in c1s]
                e_b = [e + kcoef(j, 1) * cp
                       for e, j, cp in zip(e_b, batch, cps)]
                for n in range(2, _PER):
                    cns = [2.0 * c1 * cp - cpp
                           for c1, cp, cpp in zip(c1s, cps, cpps)]
                    e_b = [e + kcoef(j, n) * cn
                           for e, j, cn in zip(e_b, batch, cns)]
                    cpps = cps
                    cps = cns
                es.extend(e_b)
            _scatter_window(acc_v, lanes, b16, j0, es)

        def window(g, c2):
            do_window(g * _G, 0)
            return c2
        carry = lax.fori_loop(0, _TCH // _G, window, carry)
        if _TCH % _G:
            do_window(_TCH - _G, _G - _TCH % _G)
        return carry

    lax.fori_loop(0, _NTC, chunk, 0)
    pltpu.sync_copy(acc_v, out_hbm.at[wid])


def _combine_body(p1_ref, p2_ref, o_ref):
    o_ref[...] = jnp.sum(p1_ref[...], axis=0) + jnp.sum(p2_ref[...], axis=0)


@jax.jit
def kernel(k_bond, eq_bond, distances, bond_batch, k_torsion, angles, torsion_batch):
    mesh = plsc.VectorSubcoreMesh(core_axis_name="c", subcore_axis_name="s")
    cp = pltpu.CompilerParams(
        needs_layout_passes=False, use_tc_tiling_on_sc=False)
    out_t = jax.ShapeDtypeStruct((_NW, _NB * _CONFS), jnp.float32)
    bond_sc = pl.kernel(
        _bond_body, out_type=out_t, mesh=mesh, compiler_params=cp,
        scratch_types=[
            pltpu.VMEM((_BC,), jnp.float32),
            pltpu.VMEM((_BC,), jnp.float32),
            pltpu.VMEM((_BC * _CONFS,), jnp.float32),
            pltpu.VMEM((_BC,), jnp.int32),
            pltpu.VMEM((_NB * _CONFS,), jnp.float32),
        ],
    )
    tors_sc = pl.kernel(
        _tors_body, out_type=out_t, mesh=mesh, compiler_params=cp,
        scratch_types=[
            pltpu.VMEM((_TCH * _PER,), jnp.float32),
            pltpu.VMEM((_TCH * _CONFS,), jnp.float32),
            pltpu.VMEM((_TCH,), jnp.int32),
            pltpu.VMEM((_NB * _CONFS,), jnp.float32),
        ],
    )
    p_bond = bond_sc(0.5 * k_bond, eq_bond, distances.reshape(-1),
                     bond_batch)
    p_tors = tors_sc(k_torsion.reshape(-1), angles.reshape(-1),
                     torsion_batch)
    total = pl.pallas_call(
        _combine_body,
        out_shape=jax.ShapeDtypeStruct((_NB, _CONFS), jnp.float32),
    )(p_bond.reshape(_NW, _NB, _CONFS), p_tors.reshape(_NW, _NB, _CONFS))
    return total


# speedup vs baseline: 1.0747x; 1.0290x over previous
import jax
import jax.numpy as jnp
from jax import lax
from jax.experimental import pallas as pl
from jax.experimental.pallas import tpu as pltpu
from jax.experimental.pallas import tpu_sc as plsc

_N_BONDS = 1600000
_N_TORS = 800000
_CONFS = 16
_NB = 64
_PER = 6
_NW = 32
_BONDS_PER_W = _N_BONDS // _NW
_TORS_PER_W = _N_TORS // _NW
_BC = 2000
_TCH = 1000
_NBC = _BONDS_PER_W // _BC
_NTC = _TORS_PER_W // _TCH
_G = 16

_COS_C = (
    9.99999992e-01,
    -4.99999918e-01,
    4.16665243e-02,
    -1.38879703e-03,
    2.47734208e-05,
    -2.71133377e-07,
    1.73689961e-09,
)
_INV_2PI = 0.15915494309189535
_PI2_HI = 6.28125
_PI2_LO = 0.0019353071795864769
_RND_MAGIC = 12582912.0


def _cos_stage(phis):
    a0, a1, a2, a3, a4, a5, a6 = _COS_C
    rns = [p * _INV_2PI for p in phis]
    nfs = [(rn + _RND_MAGIC) - _RND_MAGIC for rn in rns]
    rs = [(p - nf * _PI2_HI) - nf * _PI2_LO for p, nf in zip(phis, nfs)]
    ts = [r * r for r in rs]
    t2s = [t * t for t in ts]
    p01s = [a0 + a1 * t for t in ts]
    p23s = [a2 + a3 * t for t in ts]
    p45s = [a4 + a5 * t for t in ts]
    qs = [p45 + a6 * t2 for p45, t2 in zip(p45s, t2s)]
    lows = [p01 + t2 * p23 for p01, t2, p23 in zip(p01s, t2s, p23s)]
    return [low + (t2 * t2) * q for low, t2, q in zip(lows, t2s, qs)]


def _tree_sum(vs):
    vs = list(vs)
    while len(vs) > 1:
        vs = [vs[i] + vs[i + 1] for i in range(0, len(vs) - 1, 2)] \
            + ([vs[-1]] if len(vs) % 2 else [])
    return vs[0]


def _zero_acc(acc_v):
    zeros16 = jnp.zeros((_G,), jnp.float32)
    for i in range(_NB):
        acc_v[pl.ds(i * _G, _G)] = zeros16


def _scatter_window(acc_v, lanes, b16, j0, es):
    e_sum = _tree_sum(es)
    uniform = b16[j0] == b16[_G - 1]

    @pl.when(uniform)
    def _():
        idx = jnp.full((_G,), b16[j0]) * _G + lanes
        plsc.addupdate_scatter(acc_v, [idx], e_sum)

    @pl.when(jnp.logical_not(uniform))
    def _():
        for i, e in enumerate(es):
            idx = jnp.full((_G,), b16[j0 + i]) * _G + lanes
            plsc.addupdate_scatter(acc_v, [idx], e)


def _bond_body(kb_hbm, eq_hbm, d_hbm, bb_hbm, out_hbm,
               kb_v, eq_v, d_v, bb_v, acc_v, dma_sem):
    cid = lax.axis_index("c")
    sid = lax.axis_index("s")
    wid = sid * 2 + cid
    lanes = lax.iota(jnp.int32, _G)
    _zero_acc(acc_v)
    bond_base = wid * _BONDS_PER_W

    def chunk(ci, carry):
        base = bond_base + ci * _BC
        cps = [
            pltpu.async_copy(kb_hbm.at[pl.ds(base, _BC)], kb_v, dma_sem),
            pltpu.async_copy(eq_hbm.at[pl.ds(base, _BC)], eq_v, dma_sem),
            pltpu.async_copy(d_hbm.at[pl.ds(base * _CONFS, _BC * _CONFS)],
                             d_v, dma_sem),
            pltpu.async_copy(bb_hbm.at[pl.ds(base, _BC)], bb_v, dma_sem),
        ]
        for cpy in cps:
            cpy.wait()

        def do_window(r0):
            b16 = bb_v[pl.ds(r0, _G)]
            rsplat = jnp.full((_G,), r0, jnp.int32)
            kvs = [plsc.load_gather(kb_v, [rsplat + j]) for j in range(_G)]
            eqvs = [plsc.load_gather(eq_v, [rsplat + j]) for j in range(_G)]
            ds = [d_v[pl.ds((r0 + j) * _CONFS, _G)] for j in range(_G)]
            diffs = [d - eqv for d, eqv in zip(ds, eqvs)]
            es = [kv * (df * df) for kv, df in zip(kvs, diffs)]
            _scatter_window(acc_v, lanes, b16, 0, es)

        def window2(g, c2):
            do_window(g * (2 * _G))
            do_window(g * (2 * _G) + _G)
            return c2
        carry = lax.fori_loop(0, _BC // (2 * _G), window2, carry)
        if (_BC // _G) % 2:
            do_window(_BC - _G)
        return carry

    lax.fori_loop(0, _NBC, chunk, 0)
    pltpu.sync_copy(acc_v, out_hbm.at[wid])


def _tors_body(ktf_hbm, ang_hbm, tb_hbm, out_hbm,
               ktf_v, ang_v, tb_v, acc_v, dma_sem):
    cid = lax.axis_index("c")
    sid = lax.axis_index("s")
    wid = sid * 2 + cid
    lanes = lax.iota(jnp.int32, _G)
    _zero_acc(acc_v)
    tors_base = wid * _TORS_PER_W

    def chunk(ci, carry):
        base = tors_base + ci * _TCH
        cps = [
            pltpu.async_copy(ktf_hbm.at[pl.ds(base * _PER, _TCH * _PER)],
                             ktf_v, dma_sem),
            pltpu.async_copy(ang_hbm.at[pl.ds(base * _CONFS, _TCH * _CONFS)],
                             ang_v, dma_sem),
            pltpu.async_copy(tb_hbm.at[pl.ds(base, _TCH)], tb_v, dma_sem),
        ]
        for cpy in cps:
            cpy.wait()

        def do_window(r0, j0):
            b16 = tb_v[pl.ds(r0, _G)]
            kbase = jnp.full((_G,), r0 * _PER, jnp.int32)

            def kcoef(j, n):
                return plsc.load_gather(ktf_v, [kbase + (_PER * j + n)])

            es = []
            half = (_G - j0) // 2
            rows = list(range(j0, _G))
            for batch in (rows[:half], rows[half:]):
                phis = [ang_v[pl.ds((r0 + j) * _CONFS, _G)] for j in batch]
                c1s = _cos_stage(phis)
                e_b = [kcoef(j, 0) * c1 for j, c1 in zip(batch, c1s)]
                cpps = c1s
                cps = [2.0 * c1 * c1 - 1.0 for c1 in c1s]
                e_b = [e + kcoef(j, 1) * cp
                       for e, j, cp in zip(e_b, batch, cps)]
                for n in range(2, _PER):
                    cns = [2.0 * c1 * cp - cpp
                           for c1, cp, cpp in zip(c1s, cps, cpps)]
                    e_b = [e + kcoef(j, n) * cn
                           for e, j, cn in zip(e_b, batch, cns)]
                    cpps = cps
                    cps = cns
                es.extend(e_b)
            _scatter_window(acc_v, lanes, b16, j0, es)

        def window2(g, c2):
            do_window(g * (2 * _G), 0)
            do_window(g * (2 * _G) + _G, 0)
            return c2
        carry = lax.fori_loop(0, _TCH // (2 * _G), window2, carry)
        if (_TCH // _G) % 2:
            do_window(_TCH // _G * _G - _G, 0)
        if _TCH % _G:
            do_window(_TCH - _G, _G - _TCH % _G)
        return carry

    lax.fori_loop(0, _NTC, chunk, 0)
    pltpu.sync_copy(acc_v, out_hbm.at[wid])


def _combine_body(p1_ref, p2_ref, o_ref):
    o_ref[...] = jnp.sum(p1_ref[...], axis=0) + jnp.sum(p2_ref[...], axis=0)


@jax.jit
def kernel(k_bond, eq_bond, distances, bond_batch, k_torsion, angles, torsion_batch):
    mesh = plsc.VectorSubcoreMesh(core_axis_name="c", subcore_axis_name="s")
    cp = pltpu.CompilerParams(
        needs_layout_passes=False, use_tc_tiling_on_sc=False)
    out_t = jax.ShapeDtypeStruct((_NW, _NB * _CONFS), jnp.float32)
    bond_sc = pl.kernel(
        _bond_body, out_type=out_t, mesh=mesh, compiler_params=cp,
        scratch_types=[
            pltpu.VMEM((_BC,), jnp.float32),
            pltpu.VMEM((_BC,), jnp.float32),
            pltpu.VMEM((_BC * _CONFS,), jnp.float32),
            pltpu.VMEM((_BC,), jnp.int32),
            pltpu.VMEM((_NB * _CONFS,), jnp.float32),
            pltpu.SemaphoreType.DMA,
        ],
    )
    tors_sc = pl.kernel(
        _tors_body, out_type=out_t, mesh=mesh, compiler_params=cp,
        scratch_types=[
            pltpu.VMEM((_TCH * _PER,), jnp.float32),
            pltpu.VMEM((_TCH * _CONFS,), jnp.float32),
            pltpu.VMEM((_TCH,), jnp.int32),
            pltpu.VMEM((_NB * _CONFS,), jnp.float32),
            pltpu.SemaphoreType.DMA,
        ],
    )
    p_bond = bond_sc(0.5 * k_bond, eq_bond, distances.reshape(-1),
                     bond_batch)
    p_tors = tors_sc(k_torsion.reshape(-1), angles.reshape(-1),
                     torsion_batch)
    total = pl.pallas_call(
        _combine_body,
        out_shape=jax.ShapeDtypeStruct((_NB, _CONFS), jnp.float32),
    )(p_bond.reshape(_NW, _NB, _CONFS), p_tors.reshape(_NW, _NB, _CONFS))
    return total
